# baseline (speedup 1.0000x reference)
import jax
import jax.numpy as jnp
from jax import lax
from jax.experimental import pallas as pl
from jax.experimental.pallas import tpu as pltpu
from jax.experimental.pallas import tpu_sc as plsc

B = 8
N_PER = 1024
E_PER = 4096
T2 = N_PER + E_PER + 2
D_IN = 512
HIDDEN = 1024
LAP_K = 16
NUM_EDGE_TYPES = 7

BT = 512
NT = (T2 + BT - 1) // BT
EDGE_START = 2
NPP = N_PER + 16
EPP = (NT - EDGE_START) * BT
BH = 512
NH = HIDDEN // BH

_SC_CORES = 2
_SC_SUBCORES = 16
_NW = _SC_CORES * _SC_SUBCORES
_CHUNK = (B * E_PER) // _NW
_WPG = E_PER // _CHUNK


_NCHUNK = N_PER // _WPG
_NSUB = 128


def _sc_gather_body(gsrc_hbm, gdst_hbm, eig_hbm, nf_hbm, es_out, ed_out,
                    nft_out, idx_v, rows_v, nbuf, sem):
    wid = lax.axis_index("s") * _SC_CORES + lax.axis_index("c")
    b = wid // _WPG
    q = wid % _WPG
    in_base = wid * _CHUNK
    row_base = 2 + q * _CHUNK
    for idx_hbm, out_hbm in ((gsrc_hbm, es_out), (gdst_hbm, ed_out)):
        pltpu.sync_copy(idx_hbm.at[pl.ds(in_base, _CHUNK)], idx_v)
        pltpu.async_copy(eig_hbm.at[idx_v], rows_v, sem).wait()
        pltpu.sync_copy(rows_v, out_hbm.at[pl.ds(row_base, _CHUNK), b])
    for j in range(_NCHUNK // _NSUB):
        src0 = b * N_PER + q * _NCHUNK + j * _NSUB
        dst0 = 2 + q * _NCHUNK + j * _NSUB
        pltpu.sync_copy(nf_hbm.at[pl.ds(src0, _NSUB)], nbuf)
        pltpu.sync_copy(nbuf, nft_out.at[pl.ds(dst0, _NSUB), b])


def _sc_gather(gsrc, gdst, eigvec, node_feature):
    mesh = plsc.VectorSubcoreMesh(core_axis_name="c", subcore_axis_name="s")
    row_ty = jax.ShapeDtypeStruct((EPP, B, LAP_K), jnp.float32)
    nft_ty = jax.ShapeDtypeStruct((NPP, B, D_IN), jnp.float32)
    fn = pl.kernel(
        _sc_gather_body,
        out_type=(row_ty, row_ty, nft_ty),
        mesh=mesh,
        compiler_params=pltpu.CompilerParams(use_tc_tiling_on_sc=False),
        scratch_types=[
            pltpu.VMEM((_CHUNK,), jnp.int32),
            pltpu.VMEM((_CHUNK, LAP_K), jnp.float32),
            pltpu.VMEM((_NSUB, D_IN), jnp.float32),
            pltpu.SemaphoreType.DMA,
        ],
    )
    return fn(gsrc, gdst, eigvec, node_feature)


def _tc_body(nf_ref, eign_ref, nft_ref, eigt_ref, es_ref, ed_ref, aux_ref,
             atomw_ref, lapsum_ref, lap0_ref, lap1_ref, waux_ref,
             consts_ref, out_ref):
    f32 = jnp.float32
    s = pl.program_id(0)

    def node_part():
        x = nf_ref[...].reshape(BT * B, D_IN)
        e = eign_ref[...].reshape(BT * B, LAP_K)
        r = (
            jnp.dot(x, atomw_ref[...], preferred_element_type=f32, precision=jax.lax.Precision.DEFAULT)
            + jnp.dot(e, lapsum_ref[...], preferred_element_type=f32, precision=jax.lax.Precision.DEFAULT)
            + consts_ref[0, :][None, :]
        )
        return r.reshape(BT, B, BH)

    def node_tail():
        x = nft_ref[...].reshape(16 * B, D_IN)
        e = eigt_ref[...].reshape(16 * B, LAP_K)
        r = (
            jnp.dot(x, atomw_ref[...], preferred_element_type=f32, precision=jax.lax.Precision.DEFAULT)
            + jnp.dot(e, lapsum_ref[...], preferred_element_type=f32, precision=jax.lax.Precision.DEFAULT)
            + consts_ref[0, :][None, :]
        )
        return r.reshape(16, B, BH)

    def edge_part():
        r = (
            jnp.dot(es_ref[...].reshape(BT * B, LAP_K), lap0_ref[...],
                    preferred_element_type=f32, precision=jax.lax.Precision.DEFAULT)
            + jnp.dot(ed_ref[...].reshape(BT * B, LAP_K), lap1_ref[...],
                      preferred_element_type=f32, precision=jax.lax.Precision.DEFAULT)
            + jnp.dot(aux_ref[...].reshape(BT * B, LAP_K), waux_ref[...],
                      preferred_element_type=f32, precision=jax.lax.Precision.DEFAULT)
            + consts_ref[1, :][None, :]
        )
        return r.reshape(BT, B, BH)

    @pl.when(s < EDGE_START)
    def _():
        out_ref[...] = node_part()

    @pl.when(s == 0)
    def _():
        out_ref[0:2, :, :] = jnp.broadcast_to(
            consts_ref[2:4, :][:, None, :], (2, B, BH))

    @pl.when(s >= EDGE_START)
    def _():
        out_ref[...] = edge_part()

    @pl.when(s == EDGE_START)
    def _():
        out_ref[0:2, :, :] = node_tail()[0:2]


def kernel(node_feature, edge_index, edge_types, eigvec, atom_W, atom_b,
           edge_table, lap_W, order_table, graph_token, null_token):
    f32 = jnp.float32

    src = edge_index[0].astype(jnp.int32)
    dst = edge_index[1].astype(jnp.int32)
    goffs = (jnp.arange(B, dtype=jnp.int32) * N_PER).repeat(E_PER)
    gsrc = src + goffs
    gdst = dst + goffs

    lanes = jnp.arange(16, dtype=jnp.int32)
    onehot = (edge_types[:, None] == lanes[None, :]).astype(f32)
    order = (src == dst).astype(f32)
    aux = onehot + order[:, None] * (lanes == 7).astype(f32)[None, :]
    aux_t = jnp.pad(aux.reshape(B, E_PER, 16).transpose(1, 0, 2),
                    ((2, EPP - 2 - E_PER), (0, 0), (0, 0)))

    eign_t = jnp.pad(eigvec.reshape(B, N_PER, LAP_K).transpose(1, 0, 2),
                     ((2, NPP - 2 - N_PER), (0, 0), (0, 0)))

    lap0 = lap_W[:LAP_K]
    lap1 = lap_W[LAP_K:]
    lapsum = lap0 + lap1
    waux = jnp.zeros((16, HIDDEN), f32)
    waux = waux.at[0:NUM_EDGE_TYPES].set(edge_table)
    waux = waux.at[7].set(order_table[1] - order_table[0])
    consts = jnp.zeros((8, HIDDEN), f32)
    consts = consts.at[0].set(atom_b + order_table[1])
    consts = consts.at[1].set(order_table[0])
    consts = consts.at[2].set(graph_token[0])
    consts = consts.at[3].set(null_token[0])

    es_t, ed_t, nf_t = _sc_gather(gsrc, gdst, eigvec, node_feature)

    grid = (NT, NH)
    out = pl.pallas_call(
        _tc_body,
        grid=grid,
        in_specs=[
            pl.BlockSpec((BT, B, D_IN),
                         lambda s, h: (jnp.minimum(s, EDGE_START - 1), 0, 0)),
            pl.BlockSpec((BT, B, LAP_K),
                         lambda s, h: (jnp.minimum(s, EDGE_START - 1), 0, 0)),
            pl.BlockSpec((16, B, D_IN), lambda s, h: (N_PER // 16, 0, 0)),
            pl.BlockSpec((16, B, LAP_K), lambda s, h: (N_PER // 16, 0, 0)),
            pl.BlockSpec((BT, B, LAP_K),
                         lambda s, h: (jnp.maximum(s - EDGE_START, 0), 0, 0)),
            pl.BlockSpec((BT, B, LAP_K),
                         lambda s, h: (jnp.maximum(s - EDGE_START, 0), 0, 0)),
            pl.BlockSpec((BT, B, LAP_K),
                         lambda s, h: (jnp.maximum(s - EDGE_START, 0), 0, 0)),
            pl.BlockSpec((D_IN, BH),
                         lambda s, h: (0, jnp.where(s <= EDGE_START, h, 0))),
            pl.BlockSpec((LAP_K, BH),
                         lambda s, h: (0, jnp.where(s <= EDGE_START, h, 0))),
            pl.BlockSpec((LAP_K, BH), lambda s, h: (0, h)),
            pl.BlockSpec((LAP_K, BH), lambda s, h: (0, h)),
            pl.BlockSpec((LAP_K, BH), lambda s, h: (0, h)),
            pl.BlockSpec((8, BH), lambda s, h: (0, h)),
        ],
        out_specs=pl.BlockSpec((BT, B, BH), lambda s, h: (s, 0, h)),
        out_shape=jax.ShapeDtypeStruct((T2, B, HIDDEN), f32),
        compiler_params=pltpu.CompilerParams(
            dimension_semantics=("arbitrary", "arbitrary"),
        ),
    )(nf_t, eign_t, nf_t, eign_t, es_t, ed_t, aux_t, atom_W, lapsum,
      lap0, lap1, waux, consts)
    return out.transpose(1, 0, 2)


# speedup vs baseline: 1.0456x; 1.0456x over previous
import jax
import jax.numpy as jnp
from jax import lax
from jax.experimental import pallas as pl
from jax.experimental.pallas import tpu as pltpu
from jax.experimental.pallas import tpu_sc as plsc

B = 8
N_PER = 1024
E_PER = 4096
T2 = N_PER + E_PER + 2
D_IN = 512
HIDDEN = 1024
LAP_K = 16
NUM_EDGE_TYPES = 7

BT = 512
NT = (T2 + BT - 1) // BT
EDGE_START = 2
NET = NT - EDGE_START
NPP = N_PER + 16
EPP = NET * BT
BH = 512
NH = HIDDEN // BH

_SC_CORES = 2
_SC_SUBCORES = 16
_NW = _SC_CORES * _SC_SUBCORES
_CHUNK = (B * E_PER) // _NW
_WPG = E_PER // _CHUNK


def _sc_gather_body(gsrc_hbm, gdst_hbm, eig_hbm, es_out, ed_out,
                    idx_v, rows_v, sem):
    wid = lax.axis_index("s") * _SC_CORES + lax.axis_index("c")
    b = wid // _WPG
    q = wid % _WPG
    in_base = wid * _CHUNK
    row_base = 2 + q * _CHUNK
    for idx_hbm, out_hbm in ((gsrc_hbm, es_out), (gdst_hbm, ed_out)):
        pltpu.sync_copy(idx_hbm.at[pl.ds(in_base, _CHUNK)], idx_v)
        pltpu.async_copy(eig_hbm.at[idx_v], rows_v, sem).wait()
        pltpu.sync_copy(rows_v, out_hbm.at[pl.ds(row_base, _CHUNK), b])


def _sc_gather(gsrc, gdst, eigvec):
    mesh = plsc.VectorSubcoreMesh(core_axis_name="c", subcore_axis_name="s")
    row_ty = jax.ShapeDtypeStruct((EPP, B, LAP_K), jnp.float32)
    fn = pl.kernel(
        _sc_gather_body,
        out_type=(row_ty, row_ty),
        mesh=mesh,
        compiler_params=pltpu.CompilerParams(use_tc_tiling_on_sc=False),
        scratch_types=[
            pltpu.VMEM((_CHUNK,), jnp.int32),
            pltpu.VMEM((_CHUNK, LAP_K), jnp.float32),
            pltpu.SemaphoreType.DMA,
        ],
    )
    return fn(gsrc, gdst, eigvec)


def _edge_body(es_ref, ed_ref, aux_ref, nft_ref, eigt_ref, atomw_ref,
               lapsum_ref, lap0_ref, lap1_ref, waux_ref, consts_ref,
               out_ref):
    f32 = jnp.float32
    se = pl.program_id(0)
    prec = jax.lax.Precision.DEFAULT

    r = (
        jnp.dot(es_ref[...].reshape(BT * B, LAP_K), lap0_ref[...],
                preferred_element_type=f32, precision=prec)
        + jnp.dot(ed_ref[...].reshape(BT * B, LAP_K), lap1_ref[...],
                  preferred_element_type=f32, precision=prec)
        + jnp.dot(aux_ref[...].reshape(BT * B, LAP_K), waux_ref[...],
                  preferred_element_type=f32, precision=prec)
        + consts_ref[1, :][None, :]
    )
    out_ref[...] = r.reshape(BT, B, BH)

    @pl.when(se == 0)
    def _():
        x = nft_ref[...].reshape(8 * B, D_IN)
        e = eigt_ref[...].reshape(8 * B, LAP_K)
        rt = (
            jnp.dot(x, atomw_ref[...], preferred_element_type=f32,
                    precision=prec)
            + jnp.dot(e, lapsum_ref[...], preferred_element_type=f32,
                      precision=prec)
            + consts_ref[0, :][None, :]
        )
        out_ref[0:2, :, :] = rt.reshape(8, B, BH)[0:2]


def _node_body(nf_ref, eign_ref, atomw_ref, lapsum_ref, consts_ref,
               prev_ref, out_ref):
    f32 = jnp.float32
    s = pl.program_id(0)
    prec = jax.lax.Precision.DEFAULT

    x = nf_ref[...].reshape(BT * B, D_IN)
    e = eign_ref[...].reshape(BT * B, LAP_K)
    r = (
        jnp.dot(x, atomw_ref[...], preferred_element_type=f32,
                precision=prec)
        + jnp.dot(e, lapsum_ref[...], preferred_element_type=f32,
                  precision=prec)
        + consts_ref[0, :][None, :]
    )
    out_ref[...] = r.reshape(BT, B, BH)

    @pl.when(s == 0)
    def _():
        out_ref[0:2, :, :] = jnp.broadcast_to(
            consts_ref[2:4, :][:, None, :], (2, B, BH))


def kernel(node_feature, edge_index, edge_types, eigvec, atom_W, atom_b,
           edge_table, lap_W, order_table, graph_token, null_token):
    f32 = jnp.float32

    src = edge_index[0].astype(jnp.int32)
    dst = edge_index[1].astype(jnp.int32)
    goffs = (jnp.arange(B, dtype=jnp.int32) * N_PER).repeat(E_PER)
    gsrc = src + goffs
    gdst = dst + goffs

    lanes = jnp.arange(16, dtype=jnp.int32)
    onehot = (edge_types[:, None] == lanes[None, :]).astype(f32)
    order = (src == dst).astype(f32)
    aux = onehot + order[:, None] * (lanes == 7).astype(f32)[None, :]
    aux_t = jnp.pad(aux.reshape(B, E_PER, 16).transpose(1, 0, 2),
                    ((2, EPP - 2 - E_PER), (0, 0), (0, 0)))

    nf_t = jnp.pad(node_feature.reshape(B, N_PER, D_IN).transpose(1, 0, 2),
                   ((2, NPP - 2 - N_PER), (0, 0), (0, 0)))
    eign_t = jnp.pad(eigvec.reshape(B, N_PER, LAP_K).transpose(1, 0, 2),
                     ((2, NPP - 2 - N_PER), (0, 0), (0, 0)))

    nf_tail = jnp.pad(
        node_feature.reshape(B, N_PER, D_IN)[:, N_PER - 2:].transpose(1, 0, 2),
        ((0, 6), (0, 0), (0, 0)))
    eig_tail = jnp.pad(
        eigvec.reshape(B, N_PER, LAP_K)[:, N_PER - 2:].transpose(1, 0, 2),
        ((0, 6), (0, 0), (0, 0)))

    lap0 = lap_W[:LAP_K]
    lap1 = lap_W[LAP_K:]
    lapsum = lap0 + lap1
    waux = jnp.zeros((16, HIDDEN), f32)
    waux = waux.at[0:NUM_EDGE_TYPES].set(edge_table)
    waux = waux.at[7].set(order_table[1] - order_table[0])
    consts = jnp.zeros((8, HIDDEN), f32)
    consts = consts.at[0].set(atom_b + order_table[1])
    consts = consts.at[1].set(order_table[0])
    consts = consts.at[2].set(graph_token[0])
    consts = consts.at[3].set(null_token[0])

    es_t, ed_t = _sc_gather(gsrc, gdst, eigvec)

    edges_out = pl.pallas_call(
        _edge_body,
        grid=(NET, NH),
        in_specs=[
            pl.BlockSpec((BT, B, LAP_K), lambda se, h: (se, 0, 0)),
            pl.BlockSpec((BT, B, LAP_K), lambda se, h: (se, 0, 0)),
            pl.BlockSpec((BT, B, LAP_K), lambda se, h: (se, 0, 0)),
            pl.BlockSpec((8, B, D_IN), lambda se, h: (0, 0, 0)),
            pl.BlockSpec((8, B, LAP_K), lambda se, h: (0, 0, 0)),
            pl.BlockSpec((D_IN, BH),
                         lambda se, h: (0, jnp.where(se == 0, h, 0))),
            pl.BlockSpec((LAP_K, BH),
                         lambda se, h: (0, jnp.where(se == 0, h, 0))),
            pl.BlockSpec((LAP_K, BH), lambda se, h: (0, h)),
            pl.BlockSpec((LAP_K, BH), lambda se, h: (0, h)),
            pl.BlockSpec((LAP_K, BH), lambda se, h: (0, h)),
            pl.BlockSpec((8, BH), lambda se, h: (0, h)),
        ],
        out_specs=pl.BlockSpec((BT, B, BH),
                               lambda se, h: (se + EDGE_START, 0, h)),
        out_shape=jax.ShapeDtypeStruct((T2, B, HIDDEN), f32),
        compiler_params=pltpu.CompilerParams(
            dimension_semantics=("arbitrary", "arbitrary"),
        ),
    )(es_t, ed_t, aux_t, nf_tail, eig_tail, atom_W, lapsum, lap0, lap1,
      waux, consts)

    out = pl.pallas_call(
        _node_body,
        grid=(EDGE_START, NH),
        in_specs=[
            pl.BlockSpec((BT, B, D_IN), lambda s, h: (s, 0, 0)),
            pl.BlockSpec((BT, B, LAP_K), lambda s, h: (s, 0, 0)),
            pl.BlockSpec((D_IN, BH), lambda s, h: (0, h)),
            pl.BlockSpec((LAP_K, BH), lambda s, h: (0, h)),
            pl.BlockSpec((8, BH), lambda s, h: (0, h)),
            pl.BlockSpec(memory_space=pl.ANY),
        ],
        out_specs=pl.BlockSpec((BT, B, BH), lambda s, h: (s, 0, h)),
        out_shape=jax.ShapeDtypeStruct((T2, B, HIDDEN), f32),
        input_output_aliases={5: 0},
        compiler_params=pltpu.CompilerParams(
            dimension_semantics=("arbitrary", "arbitrary"),
        ),
    )(nf_t, eign_t, atom_W, lapsum, consts, edges_out)
    return out.transpose(1, 0, 2)


# speedup vs baseline: 1.0821x; 1.0349x over previous
import jax
import jax.numpy as jnp
from jax import lax
from jax.experimental import pallas as pl
from jax.experimental.pallas import tpu as pltpu
from jax.experimental.pallas import tpu_sc as plsc

B = 8
N_PER = 1024
E_PER = 4096
T2 = N_PER + E_PER + 2
D_IN = 512
HIDDEN = 1024
LAP_K = 16
NUM_EDGE_TYPES = 7

BT = 512
NT = (T2 + BT - 1) // BT
EDGE_START = 2
NPP = N_PER + 16
EPP = (NT - EDGE_START) * BT
BH = 512
NH = HIDDEN // BH

_SC_CORES = 2
_SC_SUBCORES = 16
_NW = _SC_CORES * _SC_SUBCORES
_CHUNK = (B * E_PER) // _NW
_WPG = E_PER // _CHUNK


def _sc_gather_body(gsrc_hbm, gdst_hbm, eig_hbm, es_out, ed_out,
                    idx_v, rows_v, sem):
    wid = lax.axis_index("s") * _SC_CORES + lax.axis_index("c")
    b = wid // _WPG
    q = wid % _WPG
    in_base = wid * _CHUNK
    row_base = 2 + q * _CHUNK
    for idx_hbm, out_hbm in ((gsrc_hbm, es_out), (gdst_hbm, ed_out)):
        pltpu.sync_copy(idx_hbm.at[pl.ds(in_base, _CHUNK)], idx_v)
        pltpu.async_copy(eig_hbm.at[idx_v], rows_v, sem).wait()
        pltpu.sync_copy(rows_v, out_hbm.at[pl.ds(row_base, _CHUNK), b])


def _sc_gather(gsrc, gdst, eigvec):
    mesh = plsc.VectorSubcoreMesh(core_axis_name="c", subcore_axis_name="s")
    row_ty = jax.ShapeDtypeStruct((EPP, B, LAP_K), jnp.float32)
    fn = pl.kernel(
        _sc_gather_body,
        out_type=(row_ty, row_ty),
        mesh=mesh,
        compiler_params=pltpu.CompilerParams(use_tc_tiling_on_sc=False),
        scratch_types=[
            pltpu.VMEM((_CHUNK,), jnp.int32),
            pltpu.VMEM((_CHUNK, LAP_K), jnp.float32),
            pltpu.SemaphoreType.DMA,
        ],
    )
    return fn(gsrc, gdst, eigvec)


def _tc_body(nf_ref, eign_ref, nft_ref, eigt_ref, es_ref, ed_ref, aux_ref,
             atomw_ref, lapsum_ref, lap0_ref, lap1_ref, waux_ref,
             consts_ref, out_ref):
    f32 = jnp.float32
    s = pl.program_id(0)
    h = pl.program_id(1)
    hs = pl.ds(h * BH, BH)
    prec = jax.lax.Precision.DEFAULT

    def node_part():
        x = nf_ref[...].reshape(BT * B, D_IN)
        e = eign_ref[...].reshape(BT * B, LAP_K)
        r = (
            jnp.dot(x, atomw_ref[:, hs], preferred_element_type=f32,
                    precision=prec)
            + jnp.dot(e, lapsum_ref[:, hs], preferred_element_type=f32,
                      precision=prec)
            + consts_ref[0, hs][None, :]
        )
        return r.reshape(BT, B, BH)

    def node_tail():
        x = nft_ref[...].reshape(16 * B, D_IN)
        e = eigt_ref[...].reshape(16 * B, LAP_K)
        r = (
            jnp.dot(x, atomw_ref[:, hs], preferred_element_type=f32,
                    precision=prec)
            + jnp.dot(e, lapsum_ref[:, hs], preferred_element_type=f32,
                      precision=prec)
            + consts_ref[0, hs][None, :]
        )
        return r.reshape(16, B, BH)

    def edge_part():
        r = (
            jnp.dot(es_ref[...].reshape(BT * B, LAP_K), lap0_ref[:, hs],
                    preferred_element_type=f32, precision=prec)
            + jnp.dot(ed_ref[...].reshape(BT * B, LAP_K), lap1_ref[:, hs],
                      preferred_element_type=f32, precision=prec)
            + jnp.dot(aux_ref[...].reshape(BT * B, LAP_K), waux_ref[:, hs],
                      preferred_element_type=f32, precision=prec)
            + consts_ref[1, hs][None, :]
        )
        return r.reshape(BT, B, BH)

    @pl.when(s < EDGE_START)
    def _():
        out_ref[...] = node_part()

    @pl.when(s == 0)
    def _():
        out_ref[0:2, :, :] = jnp.broadcast_to(
            consts_ref[2:4, hs][:, None, :], (2, B, BH))

    @pl.when(s >= EDGE_START)
    def _():
        out_ref[...] = edge_part()

    @pl.when(s == EDGE_START)
    def _():
        out_ref[0:2, :, :] = node_tail()[0:2]


def kernel(node_feature, edge_index, edge_types, eigvec, atom_W, atom_b,
           edge_table, lap_W, order_table, graph_token, null_token):
    f32 = jnp.float32

    src = edge_index[0].astype(jnp.int32)
    dst = edge_index[1].astype(jnp.int32)
    goffs = (jnp.arange(B, dtype=jnp.int32) * N_PER).repeat(E_PER)
    gsrc = src + goffs
    gdst = dst + goffs

    lanes = jnp.arange(16, dtype=jnp.int32)
    onehot = (edge_types[:, None] == lanes[None, :]).astype(f32)
    order = (src == dst).astype(f32)
    aux = onehot + order[:, None] * (lanes == 7).astype(f32)[None, :]
    aux_t = jnp.pad(aux.reshape(B, E_PER, 16).transpose(1, 0, 2),
                    ((2, EPP - 2 - E_PER), (0, 0), (0, 0)))

    nf_t = jnp.pad(node_feature.reshape(B, N_PER, D_IN).transpose(1, 0, 2),
                   ((2, NPP - 2 - N_PER), (0, 0), (0, 0)))
    eign_t = jnp.pad(eigvec.reshape(B, N_PER, LAP_K).transpose(1, 0, 2),
                     ((2, NPP - 2 - N_PER), (0, 0), (0, 0)))

    lap0 = lap_W[:LAP_K]
    lap1 = lap_W[LAP_K:]
    lapsum = lap0 + lap1
    waux = jnp.zeros((16, HIDDEN), f32)
    waux = waux.at[0:NUM_EDGE_TYPES].set(edge_table)
    waux = waux.at[7].set(order_table[1] - order_table[0])
    consts = jnp.zeros((8, HIDDEN), f32)
    consts = consts.at[0].set(atom_b + order_table[1])
    consts = consts.at[1].set(order_table[0])
    consts = consts.at[2].set(graph_token[0])
    consts = consts.at[3].set(null_token[0])

    es_t, ed_t = _sc_gather(gsrc, gdst, eigvec)

    grid = (NT, NH)
    out = pl.pallas_call(
        _tc_body,
        grid=grid,
        in_specs=[
            pl.BlockSpec((BT, B, D_IN),
                         lambda s, h: (jnp.minimum(s, EDGE_START - 1), 0, 0)),
            pl.BlockSpec((BT, B, LAP_K),
                         lambda s, h: (jnp.minimum(s, EDGE_START - 1), 0, 0)),
            pl.BlockSpec((16, B, D_IN), lambda s, h: (N_PER // 16, 0, 0)),
            pl.BlockSpec((16, B, LAP_K), lambda s, h: (N_PER // 16, 0, 0)),
            pl.BlockSpec((BT, B, LAP_K),
                         lambda s, h: (jnp.maximum(s - EDGE_START, 0), 0, 0)),
            pl.BlockSpec((BT, B, LAP_K),
                         lambda s, h: (jnp.maximum(s - EDGE_START, 0), 0, 0)),
            pl.BlockSpec((BT, B, LAP_K),
                         lambda s, h: (jnp.maximum(s - EDGE_START, 0), 0, 0)),
            pl.BlockSpec((D_IN, HIDDEN), lambda s, h: (0, 0)),
            pl.BlockSpec((LAP_K, HIDDEN), lambda s, h: (0, 0)),
            pl.BlockSpec((LAP_K, HIDDEN), lambda s, h: (0, 0)),
            pl.BlockSpec((LAP_K, HIDDEN), lambda s, h: (0, 0)),
            pl.BlockSpec((LAP_K, HIDDEN), lambda s, h: (0, 0)),
            pl.BlockSpec((8, HIDDEN), lambda s, h: (0, 0)),
        ],
        out_specs=pl.BlockSpec((BT, B, BH), lambda s, h: (s, 0, h)),
        out_shape=jax.ShapeDtypeStruct((T2, B, HIDDEN), f32),
        compiler_params=pltpu.CompilerParams(
            dimension_semantics=("arbitrary", "arbitrary"),
        ),
    )(nf_t, eign_t, nf_t, eign_t, es_t, ed_t, aux_t, atom_W, lapsum,
      lap0, lap1, waux, consts)
    return out.transpose(1, 0, 2)


# speedup vs baseline: 1.2645x; 1.1686x over previous
import jax
import jax.numpy as jnp
from jax import lax
from jax.experimental import pallas as pl
from jax.experimental.pallas import tpu as pltpu
from jax.experimental.pallas import tpu_sc as plsc

B = 8
N_PER = 1024
E_PER = 4096
T2 = N_PER + E_PER + 2
D_IN = 512
HIDDEN = 1024
LAP_K = 16
NUM_EDGE_TYPES = 7

BT = 512
NT = (T2 + BT - 1) // BT
EDGE_START = 2
NPP = N_PER + 16
EPP = (NT - EDGE_START) * BT
BH = 512
NH = HIDDEN // BH

_SC_CORES = 2
_SC_SUBCORES = 16
_NW = _SC_CORES * _SC_SUBCORES
_CHUNK = (B * E_PER) // _NW
_WPG = E_PER // _CHUNK


def _sc_gather_body(gsrc_hbm, gdst_hbm, eig_hbm, ep_out,
                    idx_v, rows_v, sem):
    wid = lax.axis_index("s") * _SC_CORES + lax.axis_index("c")
    b = wid // _WPG
    q = wid % _WPG
    in_base = wid * _CHUNK
    row_base = 2 + q * _CHUNK
    for lane0, idx_hbm in ((0, gsrc_hbm), (LAP_K, gdst_hbm)):
        pltpu.sync_copy(idx_hbm.at[pl.ds(in_base, _CHUNK)], idx_v)
        pltpu.async_copy(eig_hbm.at[idx_v], rows_v, sem).wait()
        pltpu.sync_copy(
            rows_v, ep_out.at[pl.ds(row_base, _CHUNK), b, pl.ds(lane0, LAP_K)])


def _sc_gather(gsrc, gdst, eigvec):
    mesh = plsc.VectorSubcoreMesh(core_axis_name="c", subcore_axis_name="s")
    row_ty = jax.ShapeDtypeStruct((EPP, B, 2 * LAP_K), jnp.float32)
    fn = pl.kernel(
        _sc_gather_body,
        out_type=row_ty,
        mesh=mesh,
        compiler_params=pltpu.CompilerParams(use_tc_tiling_on_sc=False),
        scratch_types=[
            pltpu.VMEM((_CHUNK,), jnp.int32),
            pltpu.VMEM((_CHUNK, LAP_K), jnp.float32),
            pltpu.SemaphoreType.DMA,
        ],
    )
    return fn(gsrc, gdst, eigvec)


def _tc_body(nf_ref, eign_ref, nft_ref, eigt_ref, ep_ref, aux_ref,
             atomw_ref, lapsum_ref, lapw_ref, waux_ref,
             consts_ref, out_ref):
    f32 = jnp.float32
    s = pl.program_id(0)
    h = pl.program_id(1)
    hs = pl.ds(h * BH, BH)
    prec = jax.lax.Precision.DEFAULT

    def node_part():
        x = nf_ref[...].reshape(BT * B, D_IN)
        e = eign_ref[...].reshape(BT * B, LAP_K)
        r = (
            jnp.dot(x, atomw_ref[:, hs], preferred_element_type=f32,
                    precision=prec)
            + jnp.dot(e, lapsum_ref[:, hs], preferred_element_type=f32,
                      precision=prec)
            + consts_ref[0, hs][None, :]
        )
        return r.reshape(BT, B, BH)

    def edge_part():
        r = (
            jnp.dot(ep_ref[...].reshape(BT * B, 2 * LAP_K), lapw_ref[:, hs],
                    preferred_element_type=f32, precision=prec)
            + jnp.dot(aux_ref[...].reshape(BT * B, LAP_K), waux_ref[:, hs],
                      preferred_element_type=f32, precision=prec)
            + consts_ref[1, hs][None, :]
        )
        return r.reshape(BT, B, BH)

    def node_tail():
        x = nft_ref[...].reshape(16 * B, D_IN)
        e = eigt_ref[...].reshape(16 * B, LAP_K)
        r = (
            jnp.dot(x, atomw_ref[:, hs], preferred_element_type=f32,
                    precision=prec)
            + jnp.dot(e, lapsum_ref[:, hs], preferred_element_type=f32,
                      precision=prec)
            + consts_ref[0, hs][None, :]
        )
        return r.reshape(16, B, BH)

    @pl.when(s < EDGE_START)
    def _():
        out_ref[...] = node_part()

    @pl.when(s == 0)
    def _():
        out_ref[0:2, :, :] = jnp.broadcast_to(
            consts_ref[2:4, hs][:, None, :], (2, B, BH))

    @pl.when(s >= EDGE_START)
    def _():
        out_ref[...] = edge_part()

    @pl.when(s == EDGE_START)
    def _():
        out_ref[0:2, :, :] = node_tail()[0:2]


def kernel(node_feature, edge_index, edge_types, eigvec, atom_W, atom_b,
           edge_table, lap_W, order_table, graph_token, null_token):
    f32 = jnp.float32

    src = edge_index[0].astype(jnp.int32)
    dst = edge_index[1].astype(jnp.int32)
    goffs = (jnp.arange(B, dtype=jnp.int32) * N_PER).repeat(E_PER)
    gsrc = src + goffs
    gdst = dst + goffs

    lanes = jnp.arange(16, dtype=jnp.int32)
    onehot = (edge_types[:, None] == lanes[None, :]).astype(f32)
    order = (src == dst).astype(f32)
    aux = onehot + order[:, None] * (lanes == 7).astype(f32)[None, :]
    aux_t = jnp.pad(aux.reshape(B, E_PER, 16).transpose(1, 0, 2),
                    ((2, EPP - 2 - E_PER), (0, 0), (0, 0)))

    nf_t = jnp.pad(node_feature.reshape(B, N_PER, D_IN).transpose(1, 0, 2),
                   ((2, NPP - 2 - N_PER), (0, 0), (0, 0)))
    eign_t = jnp.pad(eigvec.reshape(B, N_PER, LAP_K).transpose(1, 0, 2),
                     ((2, NPP - 2 - N_PER), (0, 0), (0, 0)))

    lap0 = lap_W[:LAP_K]
    lap1 = lap_W[LAP_K:]
    lapsum = lap0 + lap1
    waux = jnp.zeros((16, HIDDEN), f32)
    waux = waux.at[0:NUM_EDGE_TYPES].set(edge_table)
    waux = waux.at[7].set(order_table[1] - order_table[0])
    consts = jnp.zeros((8, HIDDEN), f32)
    consts = consts.at[0].set(atom_b + order_table[1])
    consts = consts.at[1].set(order_table[0])
    consts = consts.at[2].set(graph_token[0])
    consts = consts.at[3].set(null_token[0])

    ep_t = _sc_gather(gsrc, gdst, eigvec)

    grid = (NT, NH)
    out = pl.pallas_call(
        _tc_body,
        grid=grid,
        in_specs=[
            pl.BlockSpec((BT, B, D_IN),
                         lambda s, h: (jnp.minimum(s, EDGE_START - 1), 0, 0)),
            pl.BlockSpec((BT, B, LAP_K),
                         lambda s, h: (jnp.minimum(s, EDGE_START - 1), 0, 0)),
            pl.BlockSpec((16, B, D_IN), lambda s, h: (N_PER // 16, 0, 0)),
            pl.BlockSpec((16, B, LAP_K), lambda s, h: (N_PER // 16, 0, 0)),
            pl.BlockSpec((BT, B, 2 * LAP_K),
                         lambda s, h: (jnp.maximum(s - EDGE_START, 0), 0, 0)),
            pl.BlockSpec((BT, B, LAP_K),
                         lambda s, h: (jnp.maximum(s - EDGE_START, 0), 0, 0)),
            pl.BlockSpec((D_IN, HIDDEN), lambda s, h: (0, 0)),
            pl.BlockSpec((LAP_K, HIDDEN), lambda s, h: (0, 0)),
            pl.BlockSpec((2 * LAP_K, HIDDEN), lambda s, h: (0, 0)),
            pl.BlockSpec((LAP_K, HIDDEN), lambda s, h: (0, 0)),
            pl.BlockSpec((8, HIDDEN), lambda s, h: (0, 0)),
        ],
        out_specs=pl.BlockSpec((BT, B, BH), lambda s, h: (s, 0, h)),
        out_shape=jax.ShapeDtypeStruct((T2, B, HIDDEN), f32),
        compiler_params=pltpu.CompilerParams(
            dimension_semantics=("arbitrary", "arbitrary"),
        ),
    )(nf_t, eign_t, nf_t, eign_t, ep_t, aux_t, atom_W, lapsum,
      lap_W, waux, consts)
    return out.transpose(1, 0, 2)
